# Initial kernel scaffold; baseline (speedup 1.0000x reference)
import jax
import jax.numpy as jnp
from jax.experimental import pallas as pl


def kernel(x_lane, x_actor, edge_attr, traffic_light, is_on_route, rotate_mat, params, edge_index):
    raise NotImplementedError("write your pallas kernel here")


# speedup vs baseline: 3.5289x; 3.5289x over previous
import functools

import jax
import jax.numpy as jnp
from jax import lax
from jax.experimental import pallas as pl
from jax.experimental.pallas import tpu as pltpu
from jax.experimental.pallas import tpu_sc as plsc

NA = 10000
EDGES = 320000
D = 128
H = 8
HD = D // H
XTW = 8
GW = 16
NWORK = 32
PER = EDGES // NWORK
CH = 80
NSUB = 16
HNA = NA // 2
DCH = 40

EB = 512
EGRID = EDGES // EB
NB = 2000
NGRID = NA // NB


def _ln(x, g, b):
    mu = jnp.mean(x, axis=-1, keepdims=True)
    var = jnp.mean((x - mu) ** 2, axis=-1, keepdims=True)
    return (x - mu) / jnp.sqrt(var + 1e-5) * g + b


def _head_mats(dtype):
    row = lax.broadcasted_iota(jnp.int32, (D, H), 0) // HD
    col = lax.broadcasted_iota(jnp.int32, (D, H), 1)
    s = (row == col).astype(dtype)
    rowt = lax.broadcasted_iota(jnp.int32, (H, D), 0)
    colt = lax.broadcasted_iota(jnp.int32, (H, D), 1) // HD
    st = (rowt == colt).astype(dtype)
    return s, st


def _dot(a, b):
    return jnp.dot(a, b, preferred_element_type=jnp.float32)


def _single_mlp(x, w1, b1, g1, c1, w2, b2, g2, c2, w3, b3, g3, c3):
    h = jnp.maximum(_ln(_dot(x, w1) + b1, g1, c1), 0.0)
    h = jnp.maximum(_ln(_dot(h, w2) + b2, g2, c2), 0.0)
    return _ln(_dot(h, w3) + b3, g3, c3)


def _node_tables_body(xa, tl, rt, xl, rm,
                      n1g, n1b, qwT, qb,
                      t1, t2, t3, t4, t5, t6, t7, t8, t9, t10, t11, t12,
                      r1, r2, r3, r4, r5, r6, r7, r8, r9, r10, r11, r12,
                      q_o, ts_o, ext_o):
    x = xa[...]
    xan = _ln(x, n1g[...], n1b[...])
    q_o[...] = _dot(xan, qwT[...]) + qb[...]
    ts = _single_mlp(tl[...], t1[...], t2[...], t3[...], t4[...], t5[...],
                     t6[...], t7[...], t8[...], t9[...], t10[...], t11[...], t12[...])
    ts = ts + _single_mlp(rt[...], r1[...], r2[...], r3[...], r4[...], r5[...],
                          r6[...], r7[...], r8[...], r9[...], r10[...], r11[...], r12[...])
    ts_o[...] = ts
    red = (tl[...][:, 2:3] == 1.0).astype(jnp.float32)
    nrows = x.shape[0]
    ext_o[:, 0:4] = rm[...]
    ext_o[:, 4:6] = xl[...]
    ext_o[:, 6:7] = red
    ext_o[:, 7:8] = jnp.zeros((nrows, 1), jnp.float32)


def _branch(x0, x1, w1, b1, g1, c1, w2, b2):
    h = x0 * w1[0:1, :] + x1 * w1[1:2, :] + b1
    h = jnp.maximum(_ln(h, g1, c1), 0.0)
    return _dot(h, w2) + b2


def _edge_body(gq, gts, ext, ea,
               a1, a2, a3, a4, a5, a6,
               c1, c2, c3, c4, c5, c6,
               an1g, an1b, alwT, alb, an2g, an2b,
               kwT, kb, vwT, vb,
               num_o, ex_o):
    q = gq[...]
    ts = gts[...]
    rm00 = ext[:, 0:1]
    rm01 = ext[:, 1:2]
    rm10 = ext[:, 2:3]
    rm11 = ext[:, 3:4]
    xj0 = ext[:, 4:5]
    xj1 = ext[:, 5:6]
    red = ext[:, 6:7]
    ea0 = ea[:, 0:1]
    ea1 = ea[:, 1:2]

    xr0 = xj0 * rm00 + xj1 * rm10
    xr1 = xj0 * rm01 + xj1 * rm11
    er0 = ea0 * rm00 + ea1 * rm10
    er1 = ea0 * rm01 + ea1 * rm11

    o = _branch(xr0, xr1, a1[...], a2[...], a3[...], a4[...], a5[...], a6[...])
    o = o + _branch(er0, er1, c1[...], c2[...], c3[...], c4[...], c5[...], c6[...])
    o = jnp.maximum(_ln(o, an1g[...], an1b[...]), 0.0)
    emb = _ln(_dot(o, alwT[...]) + alb[...], an2g[...], an2b[...])
    emb = emb + ts

    k = _dot(emb, kwT[...]) + kb[...]
    v = _dot(emb, vwT[...]) + vb[...]

    s, st = _head_mats(jnp.float32)
    alpha = _dot(q * k, s) * (1.0 / (HD ** 0.5))
    alpha = jnp.where(red > 0.5, -1000000.0, alpha)
    ex = jnp.exp(alpha)
    ex128 = _dot(ex, st)
    num_o[...] = v * ex128
    ex_o[...] = ex128


def _tail_body(num, den, xa,
               n1g, n1b, ihwT, ihb, hhwT, hhb, sfwT, sfb, opwT, opb,
               n2g, n2b, m1wT, m1b, m2wT, m2b,
               out_o):
    agg = num[...] / (den[...] + 1e-16)
    x = xa[...]
    xan = _ln(x, n1g[...], n1b[...])
    gate = jax.nn.sigmoid(_dot(agg, ihwT[...]) + ihb[...]
                          + _dot(xan, hhwT[...]) + hhb[...])
    upd = agg + gate * (_dot(xan, sfwT[...]) + sfb[...] - agg)
    x2 = x + _dot(upd, opwT[...]) + opb[...]
    h = _ln(x2, n2g[...], n2b[...])
    h = _dot(jnp.maximum(_dot(h, m1wT[...]) + m1b[...], 0.0), m2wT[...]) + m2b[...]
    out_o[...] = x2 + h


def _sc_mesh():
    return plsc.VectorSubcoreMesh(core_axis_name="c", subcore_axis_name="s")


def _sc_gather(q_tab, ts_tab, ext_tab, dst_idx, src_idx):
    @functools.partial(
        pl.kernel,
        mesh=_sc_mesh(),
        compiler_params=pltpu.CompilerParams(needs_layout_passes=False),
        out_type=(jax.ShapeDtypeStruct((EDGES, D), jnp.float32),
                  jax.ShapeDtypeStruct((EDGES, D), jnp.float32),
                  jax.ShapeDtypeStruct((EDGES * GW,), jnp.float32)),
        scratch_types=[
            pltpu.VMEM((CH,), jnp.int32),
            pltpu.VMEM((CH,), jnp.int32),
            pltpu.VMEM((CH, D), jnp.float32),
            pltpu.VMEM((CH, D), jnp.float32),
            pltpu.VMEM((CH * GW,), jnp.float32),
            pltpu.VMEM((NA * XTW,), jnp.float32),
            pltpu.SemaphoreType.DMA,
            pltpu.SemaphoreType.DMA,
        ],
    )
    def k(q_h, ts_h, ext_h, dst_h, src_h, oq, ots, oe,
          idxd, idxs, qbuf, tsbuf, ebuf, ext_v, semq, semt):
        cid = lax.axis_index("c")
        sid = lax.axis_index("s")
        wid = sid * 2 + cid
        pltpu.sync_copy(ext_h, ext_v)
        base = wid * PER
        lane = lax.broadcasted_iota(jnp.int32, (16,), 0)

        def body(c, carry):
            off = base + c * CH
            pltpu.sync_copy(dst_h.at[pl.ds(off, CH)], idxd)
            pltpu.sync_copy(src_h.at[pl.ds(off, CH)], idxs)
            cpq = pltpu.async_copy(q_h.at[idxd], qbuf, semq)
            cpt = pltpu.async_copy(ts_h.at[idxs], tsbuf, semt)
            for g in range(CH // 16):
                d16 = idxd[pl.ds(g * 16, 16)] * XTW
                s16 = idxs[pl.ds(g * 16, 16)] * XTW
                e16 = (lane + g * 16) * GW
                for j in range(7):
                    n16 = d16 if j < 4 else s16
                    vals = plsc.load_gather(ext_v, [n16 + j])
                    plsc.store_scatter(ebuf, [e16 + j], vals)
            cpq.wait()
            cpt.wait()
            pltpu.sync_copy(qbuf, oq.at[pl.ds(off, CH)])
            pltpu.sync_copy(tsbuf, ots.at[pl.ds(off, CH)])
            pltpu.sync_copy(ebuf, oe.at[pl.ds(off * GW, CH * GW)])
            return carry

        lax.fori_loop(0, PER // CH, body, 0)

    return k(q_tab, ts_tab, ext_tab, dst_idx, src_idx)


def _sc_scatter(pay, idx0, idx1, iota, zeros_d):
    nzc = HNA // DCH
    @functools.partial(
        pl.kernel,
        mesh=_sc_mesh(),
        out_type=jax.ShapeDtypeStruct((NA, D), jnp.float32),
        scratch_types=[
            pltpu.VMEM((CH,), jnp.int32),
            pltpu.VMEM((CH, D), jnp.float32),
            pltpu.VMEM((DCH,), jnp.int32),
            pltpu.VMEM((DCH, D), jnp.float32),
            pltpu.VMEM_SHARED((HNA + 8, D), jnp.float32),
            pltpu.SemaphoreType.DMA,
        ],
    )
    def k(pay_h, idx0_h, idx1_h, iota_h, zd_h, out_n,
          idxv, pbuf, didx, zdbuf, accn, semn):
        cid = lax.axis_index("c")
        sid = lax.axis_index("s")
        lo = cid * HNA

        def init_one(c, carry):
            ch = sid + c * NSUB

            @pl.when(ch < nzc)
            def _():
                ds = pl.ds(ch * DCH, DCH)
                pltpu.sync_copy(iota_h.at[ds], didx)
                pltpu.sync_copy(zd_h.at[ds], zdbuf)
                pltpu.sync_copy(zdbuf, accn.at[didx])
            return carry

        lax.fori_loop(0, (nzc + NSUB - 1) // NSUB, init_one, 0)
        plsc.subcore_barrier()
        base = sid * (EDGES // NSUB)

        def body(c, carry):
            off = base + c * CH

            @pl.when(cid == 0)
            def _():
                pltpu.sync_copy(idx0_h.at[pl.ds(off, CH)], idxv)

            @pl.when(cid == 1)
            def _():
                pltpu.sync_copy(idx1_h.at[pl.ds(off, CH)], idxv)

            pltpu.sync_copy(pay_h.at[pl.ds(off, CH)], pbuf)
            pltpu.sync_copy(pbuf, accn.at[idxv], add=True)
            return carry

        lax.fori_loop(0, EDGES // NSUB // CH, body, 0)
        plsc.subcore_barrier()

        def drain_one(c, carry):
            ch = sid + c * NSUB

            @pl.when(ch < nzc)
            def _():
                ds = pl.ds(ch * DCH, DCH)
                do = pl.ds(lo + ch * DCH, DCH)
                pltpu.sync_copy(iota_h.at[ds], didx)
                pltpu.async_copy(accn.at[didx], zdbuf, semn).wait()
                pltpu.sync_copy(zdbuf, out_n.at[do])
            return carry

        lax.fori_loop(0, (nzc + NSUB - 1) // NSUB, drain_one, 0)

    return k(pay, idx0, idx1, iota, zeros_d)


def _wspec(shape):
    nd = len(shape)
    return pl.BlockSpec(shape, lambda i, _nd=nd: (0,) * _nd)


def _node_tables_call(xa, tl, rt, xl, rm, wts):
    in_arrs = [xa, tl, rt, xl, rm] + wts
    in_specs = ([pl.BlockSpec((NB, D), lambda i: (i, 0)),
                 pl.BlockSpec((NB, 4), lambda i: (i, 0)),
                 pl.BlockSpec((NB, 2), lambda i: (i, 0)),
                 pl.BlockSpec((NB, 2), lambda i: (i, 0)),
                 pl.BlockSpec((NB, 4), lambda i: (i, 0))]
                + [_wspec(w.shape) for w in wts])
    return pl.pallas_call(
        _node_tables_body,
        grid=(NGRID,),
        in_specs=in_specs,
        out_specs=(pl.BlockSpec((NB, D), lambda i: (i, 0)),
                   pl.BlockSpec((NB, D), lambda i: (i, 0)),
                   pl.BlockSpec((NB, XTW), lambda i: (i, 0))),
        out_shape=(jax.ShapeDtypeStruct((NA, D), jnp.float32),
                   jax.ShapeDtypeStruct((NA, D), jnp.float32),
                   jax.ShapeDtypeStruct((NA, XTW), jnp.float32)),
    )(*in_arrs)


def _edge_call(gq, gts, ext, ea, wts):
    in_arrs = [gq, gts, ext, ea] + wts
    in_specs = ([pl.BlockSpec((EB, D), lambda i: (i, 0)),
                 pl.BlockSpec((EB, D), lambda i: (i, 0)),
                 pl.BlockSpec((EB, GW), lambda i: (i, 0)),
                 pl.BlockSpec((EB, 2), lambda i: (i, 0))]
                + [_wspec(w.shape) for w in wts])
    return pl.pallas_call(
        _edge_body,
        grid=(EGRID,),
        in_specs=in_specs,
        out_specs=(pl.BlockSpec((EB, D), lambda i: (i, 0)),
                   pl.BlockSpec((EB, D), lambda i: (i, 0))),
        out_shape=(jax.ShapeDtypeStruct((EDGES, D), jnp.float32),
                   jax.ShapeDtypeStruct((EDGES, D), jnp.float32)),
    )(*in_arrs)


def _tail_call(num, den, xa, wts):
    in_arrs = [num, den, xa] + wts
    in_specs = ([pl.BlockSpec((NB, D), lambda i: (i, 0)),
                 pl.BlockSpec((NB, D), lambda i: (i, 0)),
                 pl.BlockSpec((NB, D), lambda i: (i, 0))]
                + [_wspec(w.shape) for w in wts])
    return pl.pallas_call(
        _tail_body,
        grid=(NGRID,),
        in_specs=in_specs,
        out_specs=pl.BlockSpec((NB, D), lambda i: (i, 0)),
        out_shape=jax.ShapeDtypeStruct((NA, D), jnp.float32),
    )(*in_arrs)


def _lin_wts(p):
    return [p["w"].T, p["b"].reshape(1, -1)]


def _ln_wts(p):
    return [p["g"].reshape(1, -1), p["b"].reshape(1, -1)]


def _single_wts(p):
    return (_lin_wts(p["l1"]) + _ln_wts(p["n1"])
            + _lin_wts(p["l2"]) + _ln_wts(p["n2"])
            + _lin_wts(p["l3"]) + _ln_wts(p["n3"]))


def _edge_wts(p):
    le = p["lane_embed"]
    b0, b1 = le["branch"][0], le["branch"][1]
    return (_lin_wts(b0["l1"]) + _ln_wts(b0["n1"]) + _lin_wts(b0["l2"])
            + _lin_wts(b1["l1"]) + _ln_wts(b1["n1"]) + _lin_wts(b1["l2"])
            + _ln_wts(le["an1"]) + _lin_wts(le["al"]) + _ln_wts(le["an2"])
            + _lin_wts(p["lin_k"]) + _lin_wts(p["lin_v"]))


def _node_wts(p):
    return (_ln_wts(p["norm1"]) + _lin_wts(p["lin_q"])
            + _single_wts(p["tl_embed"]) + _single_wts(p["route_embed"]))


def _tail_wts(p):
    return (_ln_wts(p["norm1"]) + _lin_wts(p["lin_ih"]) + _lin_wts(p["lin_hh"])
            + _lin_wts(p["lin_self"]) + _lin_wts(p["out_proj"])
            + _ln_wts(p["norm2"]) + _lin_wts(p["mlp1"]) + _lin_wts(p["mlp2"]))


def kernel(x_lane, x_actor, edge_attr, traffic_light, is_on_route, rotate_mat,
           params, edge_index):
    p = params
    src = edge_index[0]
    dst = edge_index[1]
    tl = traffic_light[:NA]
    rt = is_on_route[:NA]
    xl = x_lane[:NA]
    rm4 = rotate_mat.reshape(NA, 4)

    q_tab, ts_tab, ext_tab = _node_tables_call(x_actor, tl, rt, xl, rm4,
                                               _node_wts(p))
    gq, gts, gext_flat = _sc_gather(q_tab, ts_tab, ext_tab.reshape(-1), dst, src)
    gext = gext_flat.reshape(EDGES, GW)
    pay_num, pay_ex = _edge_call(gq, gts, gext, edge_attr, _edge_wts(p))
    zeros_d = jnp.zeros((HNA, D), jnp.float32)
    idx0 = jnp.where(dst < HNA, dst, HNA)
    idx1 = jnp.where(dst >= HNA, dst - HNA, HNA)
    iota = jnp.arange(HNA, dtype=jnp.int32)
    num = _sc_scatter(pay_num, idx0, idx1, iota, zeros_d)
    den = _sc_scatter(pay_ex, idx0, idx1, iota, zeros_d)
    return _tail_call(num, den, x_actor, _tail_wts(p))
